# baseline (speedup 1.0000x reference)
import jax
import jax.numpy as jnp
from jax import lax
from jax.experimental import pallas as pl
from jax.experimental.pallas import tpu as pltpu
from jax.experimental.pallas import tpu_sc as plsc

_NUM_CODEBOOKS = 9
_HIDDEN = 2048
_SEMANTIC = 4096
_ACOUSTIC = 2048
_N_ACOUSTIC = 8
_AUDIO_VOCAB = 20480
_STRIDE = (_AUDIO_VOCAB - _SEMANTIC - _ACOUSTIC) // (_N_ACOUSTIC - 1)
_OFFSETS = tuple(
    0 if k == 0 else _SEMANTIC + (k - 1) * _STRIDE for k in range(_NUM_CODEBOOKS)
)

_L = 16
_NC, _NS = 2, 16
_NW = _NC * _NS
_TOKENS = 4 * 4096
_TPW = _TOKENS // _NW
_TB = 16
_NB = _TPW // _TB
_GRP = _HIDDEN // (2 * _L)


def _body(ids_hbm, table_hbm, out_hbm, idxv, rows0, rows1, acc, dsem0, dsem1):
    wid = lax.axis_index("s") * _NC + lax.axis_index("c")
    base = wid * _TPW
    pltpu.sync_copy(ids_hbm.at[:, pl.ds(base, _TPW)], idxv)
    for k in range(_NUM_CODEBOOKS):
        off = _OFFSETS[k]
        if off == 0:
            continue

        def _addoff(i, carry, k=k, off=off):
            s = i * _L
            idxv[k, pl.ds(s, _L)] = idxv[k, pl.ds(s, _L)] + off
            return carry

        lax.fori_loop(0, _TPW // _L, _addoff, None)

    bufs = ((rows0, dsem0), (rows1, dsem1))

    def _gather(b, k, par):
        buf, sem = bufs[par]
        return pltpu.make_async_copy(
            table_hbm.at[idxv.at[k, pl.ds(b * _TB, _TB)]], buf, sem
        )

    _gather(0, 0, 0).start()
    _gather(0, 1, 1).start()

    def _pair(p, carry):
        for blk_i in range(2):
            b = 2 * p + blk_i
            for k in range(_NUM_CODEBOOKS):
                par = (blk_i + k) % 2
                buf, _ = bufs[par]
                _gather(b, k, par).wait()

                def _accum(g, buf=buf, first=(k == 0)):
                    sw = g * _L
                    s = g * 2 * _L
                    for t in range(_TB):
                        xw = buf[t, pl.ds(sw, _L)]
                        even = lax.bitcast_convert_type(
                            lax.shift_left(xw, jnp.int32(16)), jnp.float32
                        )
                        odd = lax.bitcast_convert_type(
                            lax.bitwise_and(xw, jnp.int32(-65536)), jnp.float32
                        )
                        if first:
                            acc[t, pl.ds(s, _L)] = even
                            acc[t, pl.ds(s + _L, _L)] = odd
                        else:
                            plsc.addupdate(acc.at[t, pl.ds(s, _L)], even)
                            plsc.addupdate(acc.at[t, pl.ds(s + _L, _L)], odd)

                plsc.parallel_loop(0, _GRP, 1, unroll=2)(_accum)
                if k < _NUM_CODEBOOKS - 2:
                    _gather(b, k + 2, par).start()
                else:
                    nk = k + 2 - _NUM_CODEBOOKS

                    @pl.when(b + 1 < _NB)
                    def _start_next(b=b, nk=nk, par=par):
                        _gather(b + 1, nk, par).start()

            pltpu.sync_copy(acc, out_hbm.at[pl.ds(base + b * _TB, _TB)])
        return carry

    lax.fori_loop(0, _NB // 2, _pair, None)


@jax.jit
def kernel(input_ids, table):
    ids2 = input_ids.reshape(_TOKENS, _NUM_CODEBOOKS).T
    tbf = (
        table.reshape(_AUDIO_VOCAB, _GRP, 2, _L)
        .transpose(0, 1, 3, 2)
        .reshape(_AUDIO_VOCAB, _HIDDEN // 2, 2)
        .astype(jnp.bfloat16)
    )
    tbf = lax.bitcast_convert_type(tbf, jnp.int32)
    out = pl.kernel(
        _body,
        out_type=jax.ShapeDtypeStruct((_TOKENS, _HIDDEN), jnp.float32),
        mesh=plsc.VectorSubcoreMesh(core_axis_name="c", subcore_axis_name="s"),
        scratch_types=[
            pltpu.VMEM((_NUM_CODEBOOKS, _TPW), jnp.int32),
            pltpu.VMEM((_TB, _HIDDEN // 2), jnp.int32),
            pltpu.VMEM((_TB, _HIDDEN // 2), jnp.int32),
            pltpu.VMEM((_TB, _HIDDEN), jnp.float32),
            pltpu.SemaphoreType.DMA,
            pltpu.SemaphoreType.DMA,
        ],
    )(ids2, tbf)
    return out.reshape(input_ids.shape[0], input_ids.shape[1], _HIDDEN)


# speedup vs baseline: 1.9420x; 1.9420x over previous
import jax
import jax.numpy as jnp
from jax import lax
from jax.experimental import pallas as pl
from jax.experimental.pallas import tpu as pltpu
from jax.experimental.pallas import tpu_sc as plsc

_NUM_CODEBOOKS = 9
_HIDDEN = 2048
_SEMANTIC = 4096
_ACOUSTIC = 2048
_N_ACOUSTIC = 8
_AUDIO_VOCAB = 20480
_STRIDE = (_AUDIO_VOCAB - _SEMANTIC - _ACOUSTIC) // (_N_ACOUSTIC - 1)
_OFFSETS = tuple(
    0 if k == 0 else _SEMANTIC + (k - 1) * _STRIDE for k in range(_NUM_CODEBOOKS)
)

_L = 16
_NC, _NS = 2, 16
_NW = _NC * _NS
_TOKENS = 4 * 4096
_TPW = _TOKENS // _NW
_TB = 16
_NB = _TPW // _TB
_W = _HIDDEN // 2
_NG = _W // _L
_TU = 8
_DEPTH = 4


def _decode(xw):
    lo = lax.bitcast_convert_type(lax.shift_left(xw, jnp.int32(16)), jnp.float32)
    hi = lax.bitcast_convert_type(
        lax.bitwise_and(xw, jnp.int32(-65536)), jnp.float32
    )
    return lo, hi


def _body(ids_hbm, table_hbm, out_hbm, idxv, b0, b1, b2, b3, acc,
          s0, s1, s2, s3):
    wid = lax.axis_index("s") * _NC + lax.axis_index("c")
    base = wid * _TPW
    pltpu.sync_copy(ids_hbm.at[:, pl.ds(base, _TPW)], idxv)
    for k in range(_NUM_CODEBOOKS):
        off = _OFFSETS[k]
        if off == 0:
            continue

        def _addoff(i, carry, k=k, off=off):
            s = i * _L
            idxv[k, pl.ds(s, _L)] = idxv[k, pl.ds(s, _L)] + off
            return carry

        lax.fori_loop(0, _TPW // _L, _addoff, None)

    bufs = ((b0, s0), (b1, s1), (b2, s2), (b3, s3))

    def _gather(b, k, u):
        buf, sem = bufs[u]
        return pltpu.make_async_copy(
            table_hbm.at[idxv.at[k, pl.ds(b * _TB, _TB)]], buf, sem
        )

    def _start_ahead(b, k, u):
        if k + _DEPTH < _NUM_CODEBOOKS:
            _gather(b, k + _DEPTH, u).start()
        else:
            nk = k + _DEPTH - _NUM_CODEBOOKS

            @pl.when(b + 1 < _NB)
            def _nxt(b=b, nk=nk, u=u):
                _gather(b + 1, nk, u).start()

    for k in range(_DEPTH):
        _gather(0, k, k).start()

    n_iter = _NG * (_TB // _TU)

    def _quad(p, carry):
        for blk_i in range(_DEPTH):
            b = _DEPTH * p + blk_i

            def u(k, blk_i=blk_i):
                return (blk_i + k) % _DEPTH

            buf0 = bufs[u(0)][0]
            _gather(b, 0, u(0)).wait()

            def _init(i, buf=buf0):
                g = i >> 1
                t0 = (i & 1) * _TU
                sw = g * _L
                for dt in range(_TU):
                    lo, hi = _decode(buf[t0 + dt, pl.ds(sw, _L)])
                    acc[t0 + dt, pl.ds(sw, _L)] = lo
                    acc[t0 + dt, pl.ds(_W + sw, _L)] = hi

            plsc.parallel_loop(0, n_iter, 1, unroll=2)(_init)
            _start_ahead(b, 0, u(0))

            for ka in (1, 3, 5, 7):
                kb = ka + 1
                bufa, bufb = bufs[u(ka)][0], bufs[u(kb)][0]
                _gather(b, ka, u(ka)).wait()
                _gather(b, kb, u(kb)).wait()

                def _accum(i, bufa=bufa, bufb=bufb):
                    g = i >> 1
                    t0 = (i & 1) * _TU
                    sw = g * _L
                    for dt in range(_TU):
                        la, ha = _decode(bufa[t0 + dt, pl.ds(sw, _L)])
                        lb, hb = _decode(bufb[t0 + dt, pl.ds(sw, _L)])
                        plsc.addupdate(acc.at[t0 + dt, pl.ds(sw, _L)], la + lb)
                        plsc.addupdate(
                            acc.at[t0 + dt, pl.ds(_W + sw, _L)], ha + hb
                        )

                plsc.parallel_loop(0, n_iter, 1, unroll=2)(_accum)
                _start_ahead(b, ka, u(ka))
                _start_ahead(b, kb, u(kb))

            pltpu.sync_copy(acc, out_hbm.at[pl.ds(base + b * _TB, _TB)])
        return carry

    lax.fori_loop(0, _NB // _DEPTH, _quad, None)


@jax.jit
def kernel(input_ids, table):
    ids2 = input_ids.reshape(_TOKENS, _NUM_CODEBOOKS).T
    lo_u = lax.bitcast_convert_type(
        table[:, :_W].astype(jnp.bfloat16), jnp.uint16
    ).astype(jnp.uint32)
    hi_u = lax.bitcast_convert_type(
        table[:, _W:].astype(jnp.bfloat16), jnp.uint16
    ).astype(jnp.uint32)
    tpk = lax.bitcast_convert_type(
        lax.bitwise_or(lax.shift_left(hi_u, jnp.uint32(16)), lo_u), jnp.int32
    )
    out = pl.kernel(
        _body,
        out_type=jax.ShapeDtypeStruct((_TOKENS, _HIDDEN), jnp.float32),
        mesh=plsc.VectorSubcoreMesh(core_axis_name="c", subcore_axis_name="s"),
        scratch_types=[
            pltpu.VMEM((_NUM_CODEBOOKS, _TPW), jnp.int32),
            pltpu.VMEM((_TB, _W), jnp.int32),
            pltpu.VMEM((_TB, _W), jnp.int32),
            pltpu.VMEM((_TB, _W), jnp.int32),
            pltpu.VMEM((_TB, _W), jnp.int32),
            pltpu.VMEM((_TB, _HIDDEN), jnp.float32),
            pltpu.SemaphoreType.DMA,
            pltpu.SemaphoreType.DMA,
            pltpu.SemaphoreType.DMA,
            pltpu.SemaphoreType.DMA,
        ],
    )(ids2, tpk)
    return out.reshape(input_ids.shape[0], input_ids.shape[1], _HIDDEN)


# speedup vs baseline: 2.0484x; 1.0548x over previous
import jax
import jax.numpy as jnp
from jax import lax
from jax.experimental import pallas as pl
from jax.experimental.pallas import tpu as pltpu
from jax.experimental.pallas import tpu_sc as plsc

_NUM_CODEBOOKS = 9
_HIDDEN = 2048
_SEMANTIC = 4096
_ACOUSTIC = 2048
_N_ACOUSTIC = 8
_AUDIO_VOCAB = 20480
_STRIDE = (_AUDIO_VOCAB - _SEMANTIC - _ACOUSTIC) // (_N_ACOUSTIC - 1)
_OFFSETS = tuple(
    0 if k == 0 else _SEMANTIC + (k - 1) * _STRIDE for k in range(_NUM_CODEBOOKS)
)

_L = 16
_NC, _NS = 2, 16
_NW = _NC * _NS
_TOKENS = 4 * 4096
_TPW = _TOKENS // _NW
_TB = 16
_NB = _TPW // _TB
_W = _HIDDEN // 2
_NG = _W // _L
_TU = 8
_DEPTH = 4


def _decode(xw):
    lo = lax.bitcast_convert_type(lax.shift_left(xw, jnp.int32(16)), jnp.float32)
    hi = lax.bitcast_convert_type(
        lax.bitwise_and(xw, jnp.int32(-65536)), jnp.float32
    )
    return lo, hi


def _body(ids_hbm, table_hbm, out_hbm, idxv, b0, b1, b2, b3, acc,
          s0, s1, s2, s3):
    wid = lax.axis_index("s") * _NC + lax.axis_index("c")
    base = wid * _TPW
    pltpu.sync_copy(ids_hbm.at[:, pl.ds(base, _TPW)], idxv)
    for k in range(_NUM_CODEBOOKS):
        off = _OFFSETS[k]
        if off == 0:
            continue

        def _addoff(i, carry, k=k, off=off):
            s = i * _L
            idxv[k, pl.ds(s, _L)] = idxv[k, pl.ds(s, _L)] + off
            return carry

        lax.fori_loop(0, _TPW // _L, _addoff, None)

    bufs = ((b0, s0), (b1, s1), (b2, s2), (b3, s3))

    def _gather(b, k, u):
        buf, sem = bufs[u]
        return pltpu.make_async_copy(
            table_hbm.at[idxv.at[k, pl.ds(b * _TB, _TB)]], buf, sem
        )

    def _start_ahead(b, k, u):
        if k + _DEPTH < _NUM_CODEBOOKS:
            _gather(b, k + _DEPTH, u).start()
        else:
            nk = k + _DEPTH - _NUM_CODEBOOKS

            @pl.when(b + 1 < _NB)
            def _nxt(b=b, nk=nk, u=u):
                _gather(b + 1, nk, u).start()

    for k in range(_DEPTH):
        _gather(0, k, k).start()

    n_iter = _NG * (_TB // _TU)

    def _quad(p, carry):
        for blk_i in range(_DEPTH):
            b = _DEPTH * p + blk_i

            def u(k, blk_i=blk_i):
                return (blk_i + k) % _DEPTH

            buf0 = bufs[u(0)][0]
            _gather(b, 0, u(0)).wait()

            def _init(i, buf=buf0):
                g = i >> 1
                t0 = (i & 1) * _TU
                sw = g * _L
                for dt in range(_TU):
                    lo, hi = _decode(buf[t0 + dt, pl.ds(sw, _L)])
                    acc[t0 + dt, pl.ds(sw, _L)] = lo
                    acc[t0 + dt, pl.ds(_W + sw, _L)] = hi

            plsc.parallel_loop(0, n_iter, 1, unroll=2)(_init)
            _start_ahead(b, 0, u(0))

            for ka in (1, 3, 5, 7):
                kb = ka + 1
                bufa, bufb = bufs[u(ka)][0], bufs[u(kb)][0]
                _gather(b, ka, u(ka)).wait()
                _gather(b, kb, u(kb)).wait()

                def _accum(i, bufa=bufa, bufb=bufb):
                    g = i >> 1
                    t0 = (i & 1) * _TU
                    sw = g * _L
                    for dt in range(_TU):
                        la, ha = _decode(bufa[t0 + dt, pl.ds(sw, _L)])
                        lb, hb = _decode(bufb[t0 + dt, pl.ds(sw, _L)])
                        plsc.addupdate(acc.at[t0 + dt, pl.ds(sw, _L)], la + lb)
                        plsc.addupdate(
                            acc.at[t0 + dt, pl.ds(_W + sw, _L)], ha + hb
                        )

                plsc.parallel_loop(0, n_iter, 1, unroll=2)(_accum)
                _start_ahead(b, ka, u(ka))
                _start_ahead(b, kb, u(kb))

            pltpu.sync_copy(acc, out_hbm.at[pl.ds(base + b * _TB, _TB)])
        return carry

    lax.fori_loop(0, _NB // _DEPTH, _quad, None)


def _pack_body(lo_ref, hi_ref, out_ref):
    lo_u = lax.bitcast_convert_type(
        lo_ref[...].astype(jnp.bfloat16), jnp.uint16
    ).astype(jnp.uint32)
    hi_u = lax.bitcast_convert_type(
        hi_ref[...].astype(jnp.bfloat16), jnp.uint16
    ).astype(jnp.uint32)
    out_ref[...] = lax.bitcast_convert_type(
        lax.bitwise_or(lax.shift_left(hi_u, jnp.uint32(16)), lo_u), jnp.int32
    )


_PACK_BM = 256


def _pack_table(table):
    return pl.pallas_call(
        _pack_body,
        out_shape=jax.ShapeDtypeStruct((_AUDIO_VOCAB, _W), jnp.int32),
        grid=(_AUDIO_VOCAB // _PACK_BM,),
        in_specs=[
            pl.BlockSpec((_PACK_BM, _W), lambda i: (i, 0)),
            pl.BlockSpec((_PACK_BM, _W), lambda i: (i, 1)),
        ],
        out_specs=pl.BlockSpec((_PACK_BM, _W), lambda i: (i, 0)),
    )(table, table)


@jax.jit
def kernel(input_ids, table):
    ids2 = input_ids.reshape(_TOKENS, _NUM_CODEBOOKS).T
    tpk = _pack_table(table)
    out = pl.kernel(
        _body,
        out_type=jax.ShapeDtypeStruct((_TOKENS, _HIDDEN), jnp.float32),
        mesh=plsc.VectorSubcoreMesh(core_axis_name="c", subcore_axis_name="s"),
        scratch_types=[
            pltpu.VMEM((_NUM_CODEBOOKS, _TPW), jnp.int32),
            pltpu.VMEM((_TB, _W), jnp.int32),
            pltpu.VMEM((_TB, _W), jnp.int32),
            pltpu.VMEM((_TB, _W), jnp.int32),
            pltpu.VMEM((_TB, _W), jnp.int32),
            pltpu.VMEM((_TB, _HIDDEN), jnp.float32),
            pltpu.SemaphoreType.DMA,
            pltpu.SemaphoreType.DMA,
            pltpu.SemaphoreType.DMA,
            pltpu.SemaphoreType.DMA,
        ],
    )(ids2, tpk)
    return out.reshape(input_ids.shape[0], input_ids.shape[1], _HIDDEN)


# speedup vs baseline: 2.1208x; 1.0353x over previous
import jax
import jax.numpy as jnp
from jax import lax
from jax.experimental import pallas as pl
from jax.experimental.pallas import tpu as pltpu
from jax.experimental.pallas import tpu_sc as plsc

_NUM_CODEBOOKS = 9
_HIDDEN = 2048
_SEMANTIC = 4096
_ACOUSTIC = 2048
_N_ACOUSTIC = 8
_AUDIO_VOCAB = 20480
_STRIDE = (_AUDIO_VOCAB - _SEMANTIC - _ACOUSTIC) // (_N_ACOUSTIC - 1)
_OFFSETS = tuple(
    0 if k == 0 else _SEMANTIC + (k - 1) * _STRIDE for k in range(_NUM_CODEBOOKS)
)

_L = 16
_NC, _NS = 2, 16
_NW = _NC * _NS
_TOKENS = 4 * 4096
_TPW = _TOKENS // _NW
_TB = 16
_NB = _TPW // _TB
_W = _HIDDEN // 2
_NG = _W // _L
_TU = 8
_DEPTH = 4


def _decode(xw):
    lo = lax.bitcast_convert_type(lax.shift_left(xw, jnp.int32(16)), jnp.float32)
    hi = lax.bitcast_convert_type(
        lax.bitwise_and(xw, jnp.int32(-65536)), jnp.float32
    )
    return lo, hi


def _body(ids_hbm, table_hbm, out_hbm, idxv, b0, b1, b2, b3, acc,
          s0, s1, s2, s3):
    wid = lax.axis_index("s") * _NC + lax.axis_index("c")
    base = wid * _TPW
    pltpu.sync_copy(ids_hbm.at[:, pl.ds(base, _TPW)], idxv)
    for k in range(_NUM_CODEBOOKS):
        off = _OFFSETS[k]
        if off == 0:
            continue

        def _addoff(i, carry, k=k, off=off):
            s = i * _L
            idxv[k, pl.ds(s, _L)] = idxv[k, pl.ds(s, _L)] + off
            return carry

        lax.fori_loop(0, _TPW // _L, _addoff, None)

    bufs = ((b0, s0), (b1, s1), (b2, s2), (b3, s3))

    def _gather(b, k, u):
        buf, sem = bufs[u]
        return pltpu.make_async_copy(
            table_hbm.at[idxv.at[k, pl.ds(b * _TB, _TB)]], buf, sem
        )

    def _start_ahead(b, k, u):
        if k + _DEPTH < _NUM_CODEBOOKS:
            _gather(b, k + _DEPTH, u).start()
        else:
            nk = k + _DEPTH - _NUM_CODEBOOKS

            @pl.when(b + 1 < _NB)
            def _nxt(b=b, nk=nk, u=u):
                _gather(b + 1, nk, u).start()

    for k in range(_DEPTH):
        _gather(0, k, k).start()

    n_iter = _NG * (_TB // _TU)

    def _quad(p, carry):
        for blk_i in range(_DEPTH):
            b = _DEPTH * p + blk_i

            def u(k, blk_i=blk_i):
                return (blk_i + k) % _DEPTH

            buf0 = bufs[u(0)][0]
            _gather(b, 0, u(0)).wait()

            def _init(i, buf=buf0):
                g = i >> 1
                t0 = (i & 1) * _TU
                sw = g * _L
                for dt in range(_TU):
                    lo, hi = _decode(buf[t0 + dt, pl.ds(sw, _L)])
                    acc[t0 + dt, pl.ds(sw, _L)] = lo
                    acc[t0 + dt, pl.ds(_W + sw, _L)] = hi

            plsc.parallel_loop(0, n_iter, 1, unroll=2)(_init)
            _start_ahead(b, 0, u(0))

            for ka in (1, 3, 5, 7):
                kb = ka + 1
                bufa, bufb = bufs[u(ka)][0], bufs[u(kb)][0]
                _gather(b, ka, u(ka)).wait()
                _gather(b, kb, u(kb)).wait()

                def _accum(i, bufa=bufa, bufb=bufb):
                    g = i >> 1
                    t0 = (i & 1) * _TU
                    sw = g * _L
                    for dt in range(_TU):
                        la, ha = _decode(bufa[t0 + dt, pl.ds(sw, _L)])
                        lb, hb = _decode(bufb[t0 + dt, pl.ds(sw, _L)])
                        plsc.addupdate(acc.at[t0 + dt, pl.ds(sw, _L)], la + lb)
                        plsc.addupdate(
                            acc.at[t0 + dt, pl.ds(_W + sw, _L)], ha + hb
                        )

                plsc.parallel_loop(0, n_iter, 1, unroll=2)(_accum)
                _start_ahead(b, ka, u(ka))
                _start_ahead(b, kb, u(kb))

            pltpu.sync_copy(acc, out_hbm.at[pl.ds(base + b * _TB, _TB)])
        return carry

    lax.fori_loop(0, _NB // _DEPTH, _quad, None)


def _pack_body(lo_ref, hi_ref, out_ref):
    lo_u = lax.bitcast_convert_type(
        lo_ref[...].astype(jnp.bfloat16), jnp.uint16
    ).astype(jnp.uint32)
    hi_u = lax.bitcast_convert_type(
        hi_ref[...].astype(jnp.bfloat16), jnp.uint16
    ).astype(jnp.uint32)
    out_ref[...] = lax.bitcast_convert_type(
        lax.bitwise_or(lax.shift_left(hi_u, jnp.uint32(16)), lo_u), jnp.int32
    )


_PACK_BM = 512


def _pack_table(table):
    return pl.pallas_call(
        _pack_body,
        out_shape=jax.ShapeDtypeStruct((_AUDIO_VOCAB, _W), jnp.int32),
        grid=(_AUDIO_VOCAB // _PACK_BM,),
        in_specs=[
            pl.BlockSpec((_PACK_BM, _W), lambda i: (i, 0)),
            pl.BlockSpec((_PACK_BM, _W), lambda i: (i, 1)),
        ],
        out_specs=pl.BlockSpec((_PACK_BM, _W), lambda i: (i, 0)),
    )(table, table)


@jax.jit
def kernel(input_ids, table):
    ids2 = input_ids.reshape(_TOKENS, _NUM_CODEBOOKS).T
    tpk = _pack_table(table)
    out = pl.kernel(
        _body,
        out_type=jax.ShapeDtypeStruct((_TOKENS, _HIDDEN), jnp.float32),
        mesh=plsc.VectorSubcoreMesh(core_axis_name="c", subcore_axis_name="s"),
        scratch_types=[
            pltpu.VMEM((_NUM_CODEBOOKS, _TPW), jnp.int32),
            pltpu.VMEM((_TB, _W), jnp.int32),
            pltpu.VMEM((_TB, _W), jnp.int32),
            pltpu.VMEM((_TB, _W), jnp.int32),
            pltpu.VMEM((_TB, _W), jnp.int32),
            pltpu.VMEM((_TB, _HIDDEN), jnp.float32),
            pltpu.SemaphoreType.DMA,
            pltpu.SemaphoreType.DMA,
            pltpu.SemaphoreType.DMA,
            pltpu.SemaphoreType.DMA,
        ],
    )(ids2, tpk)
    return out.reshape(input_ids.shape[0], input_ids.shape[1], _HIDDEN)


# speedup vs baseline: 2.1352x; 1.0068x over previous
import jax
import jax.numpy as jnp
from jax import lax
from jax.experimental import pallas as pl
from jax.experimental.pallas import tpu as pltpu
from jax.experimental.pallas import tpu_sc as plsc

_NUM_CODEBOOKS = 9
_HIDDEN = 2048
_SEMANTIC = 4096
_ACOUSTIC = 2048
_N_ACOUSTIC = 8
_AUDIO_VOCAB = 20480
_STRIDE = (_AUDIO_VOCAB - _SEMANTIC - _ACOUSTIC) // (_N_ACOUSTIC - 1)
_OFFSETS = tuple(
    0 if k == 0 else _SEMANTIC + (k - 1) * _STRIDE for k in range(_NUM_CODEBOOKS)
)

_L = 16
_NC, _NS = 2, 16
_NW = _NC * _NS
_TOKENS = 4 * 4096
_TPW = _TOKENS // _NW
_TB = 16
_NB = _TPW // _TB
_W = _HIDDEN // 2
_NG = _W // _L
_TU = 8
_DEPTH = 4


def _decode(xw):
    lo = lax.bitcast_convert_type(lax.shift_left(xw, jnp.int32(16)), jnp.float32)
    hi = lax.bitcast_convert_type(
        lax.bitwise_and(xw, jnp.int32(-65536)), jnp.float32
    )
    return lo, hi


def _body(ids_hbm, table_hbm, out_hbm, idxv, b0, b1, b2, b3, acc,
          s0, s1, s2, s3):
    wid = lax.axis_index("s") * _NC + lax.axis_index("c")
    base = wid * _TPW
    pltpu.sync_copy(ids_hbm.at[:, pl.ds(base, _TPW)], idxv)
    for k in range(_NUM_CODEBOOKS):
        off = _OFFSETS[k]
        if off == 0:
            continue

        def _addoff(i, carry, k=k, off=off):
            s = i * _L
            idxv[k, pl.ds(s, _L)] = idxv[k, pl.ds(s, _L)] + off
            return carry

        lax.fori_loop(0, _TPW // _L, _addoff, None)

    bufs = ((b0, s0), (b1, s1), (b2, s2), (b3, s3))

    def _gather(b, k, u):
        buf, sem = bufs[u]
        return pltpu.make_async_copy(
            table_hbm.at[idxv.at[k, pl.ds(b * _TB, _TB)]], buf, sem
        )

    def _start_ahead(b, k, u):
        if k + _DEPTH < _NUM_CODEBOOKS:
            _gather(b, k + _DEPTH, u).start()
        else:
            nk = k + _DEPTH - _NUM_CODEBOOKS

            @pl.when(b + 1 < _NB)
            def _nxt(b=b, nk=nk, u=u):
                _gather(b + 1, nk, u).start()

    for k in range(_DEPTH):
        _gather(0, k, k).start()

    n_iter = _NG * (_TB // _TU)

    def _quad(p, carry):
        for blk_i in range(_DEPTH):
            b = _DEPTH * p + blk_i

            def u(k, blk_i=blk_i):
                return (blk_i + k) % _DEPTH

            buf0 = bufs[u(0)][0]
            _gather(b, 0, u(0)).wait()

            def _init(i, buf=buf0):
                g = i >> 1
                t0 = (i & 1) * _TU
                sw = g * _L
                for dt in range(_TU):
                    lo, hi = _decode(buf[t0 + dt, pl.ds(sw, _L)])
                    acc[t0 + dt, pl.ds(sw, _L)] = lo
                    acc[t0 + dt, pl.ds(_W + sw, _L)] = hi

            plsc.parallel_loop(0, n_iter, 1, unroll=2)(_init)
            _start_ahead(b, 0, u(0))

            for ka in (1, 3, 5, 7):
                kb = ka + 1
                bufa, bufb = bufs[u(ka)][0], bufs[u(kb)][0]
                _gather(b, ka, u(ka)).wait()
                _gather(b, kb, u(kb)).wait()

                def _accum(i, bufa=bufa, bufb=bufb):
                    g = i >> 1
                    t0 = (i & 1) * _TU
                    sw = g * _L
                    for dt in range(_TU):
                        la, ha = _decode(bufa[t0 + dt, pl.ds(sw, _L)])
                        lb, hb = _decode(bufb[t0 + dt, pl.ds(sw, _L)])
                        plsc.addupdate(acc.at[t0 + dt, pl.ds(sw, _L)], la + lb)
                        plsc.addupdate(
                            acc.at[t0 + dt, pl.ds(_W + sw, _L)], ha + hb
                        )

                plsc.parallel_loop(0, n_iter, 1, unroll=2)(_accum)
                _start_ahead(b, ka, u(ka))
                _start_ahead(b, kb, u(kb))

            pltpu.sync_copy(acc, out_hbm.at[pl.ds(base + b * _TB, _TB)])
        return carry

    lax.fori_loop(0, _NB // _DEPTH, _quad, None)


def _pack_body(lo_ref, hi_ref, out_ref):
    lo_u = lax.bitcast_convert_type(
        lo_ref[...].astype(jnp.bfloat16), jnp.uint16
    ).astype(jnp.uint32)
    hi_u = lax.bitcast_convert_type(
        hi_ref[...].astype(jnp.bfloat16), jnp.uint16
    ).astype(jnp.uint32)
    out_ref[...] = lax.bitcast_convert_type(
        lax.bitwise_or(lax.shift_left(hi_u, jnp.uint32(16)), lo_u), jnp.int32
    )


_PACK_BM = 1024


def _pack_table(table):
    return pl.pallas_call(
        _pack_body,
        out_shape=jax.ShapeDtypeStruct((_AUDIO_VOCAB, _W), jnp.int32),
        grid=(_AUDIO_VOCAB // _PACK_BM,),
        in_specs=[
            pl.BlockSpec((_PACK_BM, _W), lambda i: (i, 0)),
            pl.BlockSpec((_PACK_BM, _W), lambda i: (i, 1)),
        ],
        out_specs=pl.BlockSpec((_PACK_BM, _W), lambda i: (i, 0)),
    )(table, table)


@jax.jit
def kernel(input_ids, table):
    ids2 = input_ids.reshape(_TOKENS, _NUM_CODEBOOKS).T
    tpk = _pack_table(table)
    out = pl.kernel(
        _body,
        out_type=jax.ShapeDtypeStruct((_TOKENS, _HIDDEN), jnp.float32),
        mesh=plsc.VectorSubcoreMesh(core_axis_name="c", subcore_axis_name="s"),
        scratch_types=[
            pltpu.VMEM((_NUM_CODEBOOKS, _TPW), jnp.int32),
            pltpu.VMEM((_TB, _W), jnp.int32),
            pltpu.VMEM((_TB, _W), jnp.int32),
            pltpu.VMEM((_TB, _W), jnp.int32),
            pltpu.VMEM((_TB, _W), jnp.int32),
            pltpu.VMEM((_TB, _HIDDEN), jnp.float32),
            pltpu.SemaphoreType.DMA,
            pltpu.SemaphoreType.DMA,
            pltpu.SemaphoreType.DMA,
            pltpu.SemaphoreType.DMA,
        ],
    )(ids2, tpk)
    return out.reshape(input_ids.shape[0], input_ids.shape[1], _HIDDEN)
